# baseline (speedup 1.0000x reference)
import functools

import jax
import jax.numpy as jnp
from jax import lax
from jax.experimental import pallas as pl
from jax.experimental.pallas import tpu as pltpu
from jax.experimental.pallas import tpu_sc as plsc

NC = 2
NS = 16
L = 16
NW = NC * NS
CH = 128


def _gmf_body(n_chunks, d, uidx_hbm, iidx_hbm, upar_hbm, ipar_hbm,
              utab, itab, wb_hbm, out_hbm,
              idx_u, idx_i, par_u, par_i, rows_u, rows_i, wb_v, out_v,
              semu, semi):
  wid = lax.axis_index("s") * NC + lax.axis_index("c")
  b_per_w = n_chunks * CH

  pltpu.sync_copy(uidx_hbm.at[wid], idx_u)
  pltpu.sync_copy(iidx_hbm.at[wid], idx_i)
  pltpu.sync_copy(upar_hbm.at[wid], par_u)
  pltpu.sync_copy(ipar_hbm.at[wid], par_i)
  pltpu.sync_copy(wb_hbm, wb_v)

  lanes = lax.iota(jnp.int32, L)
  bias = wb_v[pl.ds(d * L, L)]

  def fire(j):
    pltpu.async_copy(utab.at[idx_u.at[j]], rows_u.at[j & 1], semu)
    pltpu.async_copy(itab.at[idx_i.at[j]], rows_i.at[j & 1], semi)

  def wait(j):
    pltpu.make_async_copy(utab.at[idx_u.at[j]], rows_u.at[j & 1],
                          semu).wait()
    pltpu.make_async_copy(itab.at[idx_i.at[j]], rows_i.at[j & 1],
                          semi).wait()

  fire(0)
  for j in range(n_chunks):
    wait(j)
    if j + 1 < n_chunks:
      fire(j + 1)

    def group(g, _):
      rid = g * L + lanes
      offu = par_u[pl.ds(j * CH + g * L, L)]
      offi = par_i[pl.ds(j * CH + g * L, L)]

      def feat(f, acc):
        uvec = plsc.load_gather(rows_u.at[j & 1], [rid, offu + f])
        ivec = plsc.load_gather(rows_i.at[j & 1], [rid, offi + f])
        wvec = wb_v[pl.ds(f * L, L)]
        return acc + uvec * ivec * wvec

      acc = lax.fori_loop(0, d, feat, jnp.zeros((L,), jnp.float32))
      logits = acc + bias
      out_v[pl.ds(j * CH + g * L, L)] = 1.0 / (1.0 + jnp.exp(-logits))
      return 0

    lax.fori_loop(0, CH // L, group, 0)

  pltpu.sync_copy(out_v, out_hbm.at[pl.ds(wid * b_per_w, b_per_w)])


def kernel(user_indices, item_indices, user_table, item_table, W, b):
  batch = user_indices.shape[0]
  v, d = user_table.shape
  b_per_w = batch // NW
  n_chunks = b_per_w // CH

  utab2 = user_table.reshape(v // 2, 2 * d)
  itab2 = item_table.reshape(v // 2, 2 * d)

  ui = user_indices.astype(jnp.int32)
  ii = item_indices.astype(jnp.int32)
  uidx3 = (ui >> 1).reshape(NW, n_chunks, CH)
  iidx3 = (ii >> 1).reshape(NW, n_chunks, CH)
  upar = ((ui & 1) * d).reshape(NW, b_per_w)
  ipar = ((ii & 1) * d).reshape(NW, b_per_w)

  wb = jnp.repeat(jnp.concatenate([W[:, 0], b]), L).astype(jnp.float32)

  mesh = plsc.VectorSubcoreMesh(
      core_axis_name="c", subcore_axis_name="s", num_cores=NC, num_subcores=NS)
  run = pl.kernel(
      functools.partial(_gmf_body, n_chunks, d),
      out_type=jax.ShapeDtypeStruct((batch,), jnp.float32),
      mesh=mesh,
      compiler_params=pltpu.CompilerParams(
          needs_layout_passes=False, use_tc_tiling_on_sc=False),
      scratch_types=[
          pltpu.VMEM((n_chunks, CH), jnp.int32),
          pltpu.VMEM((n_chunks, CH), jnp.int32),
          pltpu.VMEM((b_per_w,), jnp.int32),
          pltpu.VMEM((b_per_w,), jnp.int32),
          pltpu.VMEM((2, CH, 2 * d), jnp.float32),
          pltpu.VMEM((2, CH, 2 * d), jnp.float32),
          pltpu.VMEM(((d + 1) * L,), jnp.float32),
          pltpu.VMEM((b_per_w,), jnp.float32),
          pltpu.SemaphoreType.DMA,
          pltpu.SemaphoreType.DMA,
      ],
  )
  return run(uidx3, iidx3, upar, ipar, utab2, itab2, wb)


# speedup vs baseline: 1.8654x; 1.8654x over previous
import functools

import jax
import jax.numpy as jnp
from jax import lax
from jax.experimental import pallas as pl
from jax.experimental.pallas import tpu as pltpu
from jax.experimental.pallas import tpu_sc as plsc

NC = 2
NS = 16
L = 16
NW = NC * NS
V = 1000000
D = 64
BLK = 128
NBLK = V // BLK + 1
LIM = NBLK - 1
B = 16384
PW = B // NW
SLOT = PW + L
CH = 128


def _stream_pass(t, sva, blk, tbuf, rows, lanes, tab, tail, sv2, stg,
                 semblk):
  pltpu.sync_copy(sv2.at[t], sva)
  v0 = sva[pl.ds(0, L)]
  v1 = sva[pl.ds(PW - L, L)]
  lo_b = v0[0] >> 7
  hi_end = jnp.minimum((v1[L - 1] >> 7) + 1, LIM)

  def fire(b):
    pltpu.async_copy(tab.at[:, :, pl.ds(b * BLK, BLK)], blk.at[b & 1],
                     semblk)

  def wait(b):
    pltpu.make_async_copy(tab.at[:, :, pl.ds(b * BLK, BLK)], blk.at[b & 1],
                          semblk).wait()

  def process_block(bufp, bval, ptr):
    def cstep(k, nb):
      v = sva[pl.ds(k * L, L)]
      return nb + plsc.all_reduce_population_count((v >> 7) == bval)[0]

    nb = lax.fori_loop(0, PW // L, cstep, jnp.int32(0))

    def g_body(g, z):
      src = jnp.minimum(ptr + g * L + lanes, PW - 1)
      c = plsc.load_gather(sva, [src]) & (BLK - 1)
      m = (g * L + lanes) < nb
      for f in range(D):
        fb = jnp.full((L,), f // 8, jnp.int32)
        fi = jnp.full((L,), f % 8, jnp.int32)
        vals = plsc.load_gather(bufp, [fb, fi, c])
        plsc.store_scatter(rows, [src, jnp.full((L,), f, jnp.int32)], vals,
                           mask=m)
      return z

    lax.fori_loop(0, (nb + L - 1) // L, g_body, 0)
    return ptr + nb

  @pl.when(lo_b < hi_end)
  def _():
    fire(lo_b)

  def bstep(b, ptr):
    wait(b)

    @pl.when(b + 1 < hi_end)
    def _():
      fire(b + 1)

    return process_block(blk.at[b & 1], b, ptr)

  ptr = lax.fori_loop(lo_b, hi_end, bstep, jnp.int32(0))

  pltpu.sync_copy(tail, tbuf)
  process_block(tbuf, jnp.int32(LIM), ptr)

  pltpu.sync_copy(rows, stg.at[pl.ds(t * PW, PW)])


def _gmf_stream_body(utab, itab, utail, itail, svu2, svi2, su_st, si_st,
                     sva, blk, tbuf, rows, semblk):
  t = lax.axis_index("s") * NC + lax.axis_index("c")
  lanes = lax.iota(jnp.int32, L)
  _stream_pass(t, sva, blk, tbuf, rows, lanes, utab, utail, svu2, su_st,
               semblk)
  _stream_pass(t, sva, blk, tbuf, rows, lanes, itab, itail, svi2, si_st,
               semblk)


def _gmf_dot_body(n_chunks, su_st, si_st, pos_u, pos_i, wb_hbm, out_hbm,
                  idx_u, idx_i, rows_u, rows_i, wb_v, out_v, sem):
  wid = lax.axis_index("s") * NC + lax.axis_index("c")

  pltpu.sync_copy(pos_u.at[wid], idx_u)
  pltpu.sync_copy(pos_i.at[wid], idx_i)
  pltpu.sync_copy(wb_hbm, wb_v)

  copies = []
  for j in range(n_chunks):
    copies.append(
        pltpu.async_copy(su_st.at[idx_u.at[j]], rows_u.at[pl.ds(j * CH, CH)],
                         sem))
    copies.append(
        pltpu.async_copy(si_st.at[idx_i.at[j]], rows_i.at[pl.ds(j * CH, CH)],
                         sem))
  for c in copies:
    c.wait()

  lanes = lax.iota(jnp.int32, L)
  bias = wb_v[pl.ds(D * L, L)]

  def group(g, _):
    rid = g * L + lanes

    def feat(f, acc):
      col = jnp.full((L,), f, jnp.int32)
      uvec = plsc.load_gather(rows_u, [rid, col])
      ivec = plsc.load_gather(rows_i, [rid, col])
      wvec = wb_v[pl.ds(f * L, L)]
      return acc + uvec * ivec * wvec

    acc = lax.fori_loop(0, D, feat, jnp.zeros((L,), jnp.float32))
    out_v[pl.ds(g * L, L)] = 1.0 / (1.0 + jnp.exp(-(acc + bias)))
    return 0

  lax.fori_loop(0, PW // L, group, 0)
  pltpu.sync_copy(out_v, out_hbm.at[pl.ds(wid * PW, PW)])


def kernel(user_indices, item_indices, user_table, item_table, W, b):
  utab3 = user_table.T.reshape(8, 8, V)
  itab3 = item_table.T.reshape(8, 8, V)
  pad = ((0, 0), (0, BLK - (V - LIM * BLK)))
  utail = jnp.pad(user_table[LIM * BLK:].T, pad).reshape(8, 8, BLK)
  itail = jnp.pad(item_table[LIM * BLK:].T, pad).reshape(8, 8, BLK)

  ui = user_indices.astype(jnp.int32)
  ii = item_indices.astype(jnp.int32)
  osu = jnp.argsort(ui)
  osi = jnp.argsort(ii)
  svu2 = ui[osu].reshape(NW, PW)
  svi2 = ii[osi].reshape(NW, PW)
  ar = jnp.arange(B, dtype=jnp.int32)
  pu = jnp.zeros((B,), jnp.int32).at[osu].set(ar)
  pi = jnp.zeros((B,), jnp.int32).at[osi].set(ar)
  pos_u = pu.reshape(NW, PW // CH, CH)
  pos_i = pi.reshape(NW, PW // CH, CH)

  wb = jnp.repeat(jnp.concatenate([W[:, 0], b]), L).astype(jnp.float32)

  mesh = plsc.VectorSubcoreMesh(
      core_axis_name="c", subcore_axis_name="s", num_cores=NC, num_subcores=NS)
  params = pltpu.CompilerParams(needs_layout_passes=False)

  stream = pl.kernel(
      _gmf_stream_body,
      out_type=(
          jax.ShapeDtypeStruct((B, D), jnp.float32),
          jax.ShapeDtypeStruct((B, D), jnp.float32),
      ),
      mesh=mesh,
      compiler_params=params,
      scratch_types=[
          pltpu.VMEM((PW,), jnp.int32),
          pltpu.VMEM((2, 8, 8, BLK), jnp.float32),
          pltpu.VMEM((8, 8, BLK), jnp.float32),
          pltpu.VMEM((PW, D), jnp.float32),
          pltpu.SemaphoreType.DMA,
      ],
  )
  su_st, si_st = stream(utab3, itab3, utail, itail, svu2, svi2)

  dot = pl.kernel(
      functools.partial(_gmf_dot_body, PW // CH),
      out_type=jax.ShapeDtypeStruct((B,), jnp.float32),
      mesh=mesh,
      compiler_params=pltpu.CompilerParams(
          needs_layout_passes=False, use_tc_tiling_on_sc=False),
      scratch_types=[
          pltpu.VMEM((PW // CH, CH), jnp.int32),
          pltpu.VMEM((PW // CH, CH), jnp.int32),
          pltpu.VMEM((PW, D), jnp.float32),
          pltpu.VMEM((PW, D), jnp.float32),
          pltpu.VMEM(((D + 1) * L,), jnp.float32),
          pltpu.VMEM((PW,), jnp.float32),
          pltpu.SemaphoreType.DMA,
      ],
  )
  return dot(su_st, si_st, pos_u, pos_i, wb)
